# Initial kernel scaffold; baseline (speedup 1.0000x reference)
import functools

import jax
import jax.numpy as jnp
from jax import lax
from jax.experimental import pallas as pl
from jax.experimental.pallas import tpu as pltpu
from jax.experimental.pallas import tpu_sc as plsc

NN = 10000
NP = 10240
EE = 320000
FX = 128
FE = 16
FO = 128
EPS = 1e-5
SLOPE = 0.1

CH = 128
NCHUNK = EE // CH
TPS = NP // 16


def _lrelu(v):
    return jnp.where(v >= 0.0, v, SLOPE * v)


def _sc_gather(p_pad, row):
    info = plsc.get_sparse_core_info()
    nc, ns = info.num_cores, info.num_subcores
    nw = nc * ns
    iters = (NCHUNK + nw - 1) // nw

    @functools.partial(
        pl.kernel,
        mesh=plsc.VectorSubcoreMesh(core_axis_name="c", subcore_axis_name="s"),
        out_type=jax.ShapeDtypeStruct((EE, FO), jnp.float32),
        scratch_types=[
            pltpu.VMEM((2, CH), jnp.int32),
            pltpu.VMEM((2, CH, FO), jnp.float32),
            pltpu.SemaphoreType.DMA,
        ],
    )
    def k(p_hbm, row_hbm, out_hbm, idx_v, rows_v, sem):
        cid = lax.axis_index("c")
        sid = lax.axis_index("s")
        wid = sid * nc + cid

        def body(j, carry):
            c = wid + j * nw

            @pl.when(c < NCHUNK)
            def _():
                b = j % 2
                base = c * CH
                pltpu.sync_copy(row_hbm.at[pl.ds(base, CH)], idx_v.at[b])
                pltpu.async_copy(p_hbm.at[idx_v.at[b]], rows_v.at[b], sem).wait()
                pltpu.sync_copy(rows_v.at[b], out_hbm.at[pl.ds(base, CH)])

            return carry

        lax.fori_loop(0, iters, body, 0)

    return k(p_pad, row)


def _sc_scatter(z2, col):
    info = plsc.get_sparse_core_info()
    nc, ns = info.num_cores, info.num_subcores
    nw = nc * ns
    iters = (NCHUNK + nw - 1) // nw

    @functools.partial(
        pl.kernel,
        mesh=plsc.VectorSubcoreMesh(core_axis_name="c", subcore_axis_name="s"),
        out_type=(
            jax.ShapeDtypeStruct((2, NP, FO), jnp.float32),
            jax.ShapeDtypeStruct((2, NP), jnp.float32),
        ),
        scratch_types=[
            pltpu.VMEM((2, CH), jnp.int32),
            pltpu.VMEM((2, CH, FO), jnp.float32),
            pltpu.VMEM((CH, FO), jnp.float32),
            pltpu.VMEM((TPS,), jnp.float32),
            pltpu.VMEM((CH,), jnp.float32),
            pltpu.VMEM_SHARED((NP, FO), jnp.float32),
            pltpu.VMEM_SHARED((NP,), jnp.float32),
        ],
    )
    def k(z2_hbm, col_hbm, sum_hbm, cnt_hbm, idx_v, rows_v, zblk, zcnt,
          ones_v, acc, acc_cnt):
        cid = lax.axis_index("c")
        sid = lax.axis_index("s")
        wid = sid * nc + cid
        zf = jnp.zeros((16,), jnp.float32)
        of = jnp.ones((16,), jnp.float32)

        def zero_blk(i, carry):
            zblk[i // 8, pl.ds((i % 8) * 16, 16)] = zf
            return carry

        lax.fori_loop(0, CH * FO // 16, zero_blk, 0)

        def zero_cnt(i, carry):
            zcnt[pl.ds(i * 16, 16)] = zf
            return carry

        lax.fori_loop(0, TPS // 16, zero_cnt, 0)
        for i in range(CH // 16):
            ones_v[pl.ds(i * 16, 16)] = of

        for i in range(TPS // CH):
            pltpu.sync_copy(zblk, acc.at[pl.ds(sid * TPS + i * CH, CH)])
        pltpu.sync_copy(zcnt, acc_cnt.at[pl.ds(sid * TPS, TPS)])
        plsc.subcore_barrier()

        def body(j, carry):
            c = wid + j * nw

            @pl.when(c < NCHUNK)
            def _():
                b = j % 2
                base = c * CH
                pltpu.sync_copy(col_hbm.at[pl.ds(base, CH)], idx_v.at[b])
                pltpu.sync_copy(z2_hbm.at[pl.ds(base, CH)], rows_v.at[b])
                pltpu.sync_copy(rows_v.at[b], acc.at[idx_v.at[b]], add=True)
                pltpu.sync_copy(ones_v, acc_cnt.at[idx_v.at[b]], add=True)

            return carry

        lax.fori_loop(0, iters, body, 0)
        plsc.subcore_barrier()

        pltpu.sync_copy(acc.at[pl.ds(sid * TPS, TPS)],
                        sum_hbm.at[cid].at[pl.ds(sid * TPS, TPS)])
        pltpu.sync_copy(acc_cnt.at[pl.ds(sid * TPS, TPS)],
                        cnt_hbm.at[cid].at[pl.ds(sid * TPS, TPS)])

    return k(z2, col)


def _tc_pre(x_pad, wx, b):
    br = 1280

    def body(x_ref, w_ref, b_ref, o_ref):
        o_ref[...] = (
            jnp.dot(x_ref[...], w_ref[...], preferred_element_type=jnp.float32)
            + b_ref[...]
        )

    return pl.pallas_call(
        body,
        grid=(NP // br,),
        in_specs=[
            pl.BlockSpec((br, FX), lambda i: (i, 0)),
            pl.BlockSpec((FX, FO), lambda i: (0, 0)),
            pl.BlockSpec((1, FO), lambda i: (0, 0)),
        ],
        out_specs=pl.BlockSpec((br, FO), lambda i: (i, 0)),
        out_shape=jax.ShapeDtypeStruct((NP, FO), jnp.float32),
    )(x_pad, wx, b.reshape(1, FO))


def _tc_edge(gathered, ea, wae, wb, bbias):
    be = 4000

    def body(g_ref, e_ref, wae_ref, wb_ref, b_ref, o_ref):
        h = g_ref[...] + jnp.dot(
            e_ref[...], wae_ref[...], preferred_element_type=jnp.float32
        )
        h = _lrelu(h)
        h = jnp.dot(h, wb_ref[...], preferred_element_type=jnp.float32) + b_ref[...]
        o_ref[...] = _lrelu(h)

    return pl.pallas_call(
        body,
        grid=(EE // be,),
        in_specs=[
            pl.BlockSpec((be, FO), lambda i: (i, 0)),
            pl.BlockSpec((be, FE), lambda i: (i, 0)),
            pl.BlockSpec((FE, FO), lambda i: (0, 0)),
            pl.BlockSpec((FO, FO), lambda i: (0, 0)),
            pl.BlockSpec((1, FO), lambda i: (0, 0)),
        ],
        out_specs=pl.BlockSpec((be, FO), lambda i: (i, 0)),
        out_shape=jax.ShapeDtypeStruct((EE, FO), jnp.float32),
    )(gathered, ea, wae, wb, bbias.reshape(1, FO))


def _tc_node(s_part, c_part, x_pad, wc, bc, wax, wag, ba, wb, bb2, wc2, bc2):
    br = 1280
    c3 = c_part.reshape(2, NP, 1)

    def body(s_ref, c_ref, x_ref, wc_ref, bc_ref, wax_ref, wag_ref, ba_ref,
             wb_ref, bb_ref, wc2_ref, bc2_ref, o_ref):
        s = s_ref[0] + s_ref[1]
        cnt = c_ref[0] + c_ref[1]
        sums = (
            jnp.dot(s, wc_ref[...], preferred_element_type=jnp.float32)
            + cnt * bc_ref[...]
        )
        agg = sums / jnp.maximum(cnt, 1.0)
        h = (
            jnp.dot(x_ref[...], wax_ref[...], preferred_element_type=jnp.float32)
            + jnp.dot(agg, wag_ref[...], preferred_element_type=jnp.float32)
            + ba_ref[...]
        )
        h = _lrelu(h)
        h = jnp.dot(h, wb_ref[...], preferred_element_type=jnp.float32) + bb_ref[...]
        h = _lrelu(h)
        o_ref[...] = (
            jnp.dot(h, wc2_ref[...], preferred_element_type=jnp.float32)
            + bc2_ref[...]
        )

    return pl.pallas_call(
        body,
        grid=(NP // br,),
        in_specs=[
            pl.BlockSpec((2, br, FO), lambda i: (0, i, 0)),
            pl.BlockSpec((2, br, 1), lambda i: (0, i, 0)),
            pl.BlockSpec((br, FX), lambda i: (i, 0)),
            pl.BlockSpec((FO, FO), lambda i: (0, 0)),
            pl.BlockSpec((1, FO), lambda i: (0, 0)),
            pl.BlockSpec((FX, FO), lambda i: (0, 0)),
            pl.BlockSpec((FO, FO), lambda i: (0, 0)),
            pl.BlockSpec((1, FO), lambda i: (0, 0)),
            pl.BlockSpec((FO, FO), lambda i: (0, 0)),
            pl.BlockSpec((1, FO), lambda i: (0, 0)),
            pl.BlockSpec((FO, FO), lambda i: (0, 0)),
            pl.BlockSpec((1, FO), lambda i: (0, 0)),
        ],
        out_specs=pl.BlockSpec((br, FO), lambda i: (i, 0)),
        out_shape=jax.ShapeDtypeStruct((NP, FO), jnp.float32),
    )(s_part, c3, x_pad, wc, bc.reshape(1, FO), wax, wag, ba.reshape(1, FO),
      wb, bb2.reshape(1, FO), wc2, bc2.reshape(1, FO))


def _fold(g, bb_, rm, rv, w, lb):
    s = g * lax.rsqrt(rv + EPS)
    t = bb_ - rm * s
    return w * s[:, None], t @ w + lb


def kernel(x, edge_index, edge_attr, u, batch, g1a, bb1a, rm1a, rv1a, w1a,
           lb1a, g1b, bb1b, rm1b, rv1b, w1b, lb1b, g1c, bb1c, rm1c, rv1c,
           w1c, lb1c, g2a, bb2a, rm2a, rv2a, w2a, lb2a, g2b, bb2b, rm2b,
           rv2b, w2b, lb2b, g2c, bb2c, rm2c, rv2c, w2c, lb2c):
    w1a_f, b1a_f = _fold(g1a, bb1a, rm1a, rv1a, w1a, lb1a)
    w1b_f, b1b_f = _fold(g1b, bb1b, rm1b, rv1b, w1b, lb1b)
    w1c_f, b1c_f = _fold(g1c, bb1c, rm1c, rv1c, w1c, lb1c)
    w2a_f, b2a_f = _fold(g2a, bb2a, rm2a, rv2a, w2a, lb2a)
    w2b_f, b2b_f = _fold(g2b, bb2b, rm2b, rv2b, w2b, lb2b)
    w2c_f, b2c_f = _fold(g2c, bb2c, rm2c, rv2c, w2c, lb2c)
    w1ax, w1ae = w1a_f[:FX], w1a_f[FX:]
    w2ax, w2ag = w2a_f[:FX], w2a_f[FX:]

    row = edge_index[0].astype(jnp.int32)
    col = edge_index[1].astype(jnp.int32)
    x_pad = jnp.pad(x, ((0, NP - NN), (0, 0)))

    p = _tc_pre(x_pad, w1ax, b1a_f)
    gathered = _sc_gather(p, row)
    z2 = _tc_edge(gathered, edge_attr, w1ae, w1b_f, b1b_f)
    s_part, c_part = _sc_scatter(z2, col)
    out = _tc_node(s_part, c_part, x_pad, w1c_f, b1c_f, w2ax, w2ag, b2a_f,
                   w2b_f, b2b_f, w2c_f, b2c_f)
    return out[:NN]


# speedup vs baseline: 3.4095x; 3.4095x over previous
import functools

import jax
import jax.numpy as jnp
from jax import lax
from jax.experimental import pallas as pl
from jax.experimental.pallas import tpu as pltpu
from jax.experimental.pallas import tpu_sc as plsc

NN = 10000
NP = 10240
EE = 320000
FX = 128
FE = 16
FO = 128
EPS = 1e-5
SLOPE = 0.1

CH = 128
NCHUNK = EE // CH
TPS = NP // 16


def _lrelu(v):
    return jnp.where(v >= 0.0, v, SLOPE * v)


def _sc_gather(p_pad, row):
    info = plsc.get_sparse_core_info()
    nc, ns = info.num_cores, info.num_subcores
    nw = nc * ns
    iters = (NCHUNK + nw - 1) // nw

    @functools.partial(
        pl.kernel,
        mesh=plsc.VectorSubcoreMesh(core_axis_name="c", subcore_axis_name="s"),
        out_type=jax.ShapeDtypeStruct((EE, FO), jnp.float32),
        scratch_types=[
            pltpu.VMEM((2, CH), jnp.int32),
            pltpu.VMEM((2, CH, FO), jnp.float32),
            pltpu.SemaphoreType.DMA,
        ],
    )
    def k(p_hbm, row_hbm, out_hbm, idx_v, rows_v, sem):
        cid = lax.axis_index("c")
        sid = lax.axis_index("s")
        wid = sid * nc + cid

        def body(j, carry):
            c = wid + j * nw

            @pl.when(c < NCHUNK)
            def _():
                b = j % 2
                base = c * CH
                pltpu.sync_copy(row_hbm.at[pl.ds(base, CH)], idx_v.at[b])
                pltpu.async_copy(p_hbm.at[idx_v.at[b]], rows_v.at[b], sem).wait()
                pltpu.sync_copy(rows_v.at[b], out_hbm.at[pl.ds(base, CH)])

            return carry

        lax.fori_loop(0, iters, body, 0)

    return k(p_pad, row)


def _sc_scatter(z2, col):
    info = plsc.get_sparse_core_info()
    nc, ns = info.num_cores, info.num_subcores
    nw = nc * ns
    iters = (NCHUNK + nw - 1) // nw

    @functools.partial(
        pl.kernel,
        mesh=plsc.VectorSubcoreMesh(core_axis_name="c", subcore_axis_name="s"),
        out_type=(
            jax.ShapeDtypeStruct((2, NP, FO), jnp.float32),
            jax.ShapeDtypeStruct((2, NP), jnp.float32),
        ),
        scratch_types=[
            pltpu.VMEM((2, CH), jnp.int32),
            pltpu.VMEM((2, CH, FO), jnp.float32),
            pltpu.VMEM((TPS,), jnp.float32),
            pltpu.VMEM((CH,), jnp.float32),
            pltpu.VMEM_SHARED((NP, FO), jnp.float32),
            pltpu.VMEM_SHARED((NP,), jnp.float32),
        ],
    )
    def k(z2_hbm, col_hbm, sum_hbm, cnt_hbm, idx_v, rows_v, zcnt,
          ones_v, acc, acc_cnt):
        cid = lax.axis_index("c")
        sid = lax.axis_index("s")
        wid = sid * nc + cid
        zf = jnp.zeros((16,), jnp.float32)
        of = jnp.ones((16,), jnp.float32)

        def zero_blk(i, carry):
            rows_v[0, i // 8, pl.ds((i % 8) * 16, 16)] = zf
            return carry

        lax.fori_loop(0, CH * FO // 16, zero_blk, 0)

        def zero_cnt(i, carry):
            zcnt[pl.ds(i * 16, 16)] = zf
            return carry

        lax.fori_loop(0, TPS // 16, zero_cnt, 0)
        for i in range(CH // 16):
            ones_v[pl.ds(i * 16, 16)] = of

        for i in range(TPS // CH):
            pltpu.sync_copy(rows_v.at[0], acc.at[pl.ds(sid * TPS + i * CH, CH)])
        pltpu.sync_copy(zcnt, acc_cnt.at[pl.ds(sid * TPS, TPS)])
        plsc.subcore_barrier()

        def body(j, carry):
            c = wid + j * nw

            @pl.when(c < NCHUNK)
            def _():
                b = j % 2
                base = c * CH
                pltpu.sync_copy(col_hbm.at[pl.ds(base, CH)], idx_v.at[b])
                pltpu.sync_copy(z2_hbm.at[pl.ds(base, CH)], rows_v.at[b])
                pltpu.sync_copy(rows_v.at[b], acc.at[idx_v.at[b]], add=True)
                pltpu.sync_copy(ones_v, acc_cnt.at[idx_v.at[b]], add=True)

            return carry

        lax.fori_loop(0, iters, body, 0)
        plsc.subcore_barrier()

        pltpu.sync_copy(acc.at[pl.ds(sid * TPS, TPS)],
                        sum_hbm.at[cid].at[pl.ds(sid * TPS, TPS)])
        pltpu.sync_copy(acc_cnt.at[pl.ds(sid * TPS, TPS)],
                        cnt_hbm.at[cid].at[pl.ds(sid * TPS, TPS)])

    return k(z2, col)


def _tc_pre(x_pad, wx, b):
    br = 1280

    def body(x_ref, w_ref, b_ref, o_ref):
        o_ref[...] = (
            jnp.dot(x_ref[...], w_ref[...], preferred_element_type=jnp.float32)
            + b_ref[...]
        )

    return pl.pallas_call(
        body,
        grid=(NP // br,),
        in_specs=[
            pl.BlockSpec((br, FX), lambda i: (i, 0)),
            pl.BlockSpec((FX, FO), lambda i: (0, 0)),
            pl.BlockSpec((1, FO), lambda i: (0, 0)),
        ],
        out_specs=pl.BlockSpec((br, FO), lambda i: (i, 0)),
        out_shape=jax.ShapeDtypeStruct((NP, FO), jnp.float32),
    )(x_pad, wx, b.reshape(1, FO))


def _tc_edge(gathered, ea, wae, wb, bbias):
    be = 4000

    def body(g_ref, e_ref, wae_ref, wb_ref, b_ref, o_ref):
        h = g_ref[...] + jnp.dot(
            e_ref[...], wae_ref[...], preferred_element_type=jnp.float32
        )
        h = _lrelu(h)
        h = jnp.dot(h, wb_ref[...], preferred_element_type=jnp.float32) + b_ref[...]
        o_ref[...] = _lrelu(h)

    return pl.pallas_call(
        body,
        grid=(EE // be,),
        in_specs=[
            pl.BlockSpec((be, FO), lambda i: (i, 0)),
            pl.BlockSpec((be, FE), lambda i: (i, 0)),
            pl.BlockSpec((FE, FO), lambda i: (0, 0)),
            pl.BlockSpec((FO, FO), lambda i: (0, 0)),
            pl.BlockSpec((1, FO), lambda i: (0, 0)),
        ],
        out_specs=pl.BlockSpec((be, FO), lambda i: (i, 0)),
        out_shape=jax.ShapeDtypeStruct((EE, FO), jnp.float32),
    )(gathered, ea, wae, wb, bbias.reshape(1, FO))


def _tc_node(s_part, c_part, x_pad, wc, bc, wax, wag, ba, wb, bb2, wc2, bc2):
    br = 1280
    c3 = c_part.reshape(2, NP, 1)

    def body(s_ref, c_ref, x_ref, wc_ref, bc_ref, wax_ref, wag_ref, ba_ref,
             wb_ref, bb_ref, wc2_ref, bc2_ref, o_ref):
        s = s_ref[0] + s_ref[1]
        cnt = c_ref[0] + c_ref[1]
        sums = (
            jnp.dot(s, wc_ref[...], preferred_element_type=jnp.float32)
            + cnt * bc_ref[...]
        )
        agg = sums / jnp.maximum(cnt, 1.0)
        h = (
            jnp.dot(x_ref[...], wax_ref[...], preferred_element_type=jnp.float32)
            + jnp.dot(agg, wag_ref[...], preferred_element_type=jnp.float32)
            + ba_ref[...]
        )
        h = _lrelu(h)
        h = jnp.dot(h, wb_ref[...], preferred_element_type=jnp.float32) + bb_ref[...]
        h = _lrelu(h)
        o_ref[...] = (
            jnp.dot(h, wc2_ref[...], preferred_element_type=jnp.float32)
            + bc2_ref[...]
        )

    return pl.pallas_call(
        body,
        grid=(NP // br,),
        in_specs=[
            pl.BlockSpec((2, br, FO), lambda i: (0, i, 0)),
            pl.BlockSpec((2, br, 1), lambda i: (0, i, 0)),
            pl.BlockSpec((br, FX), lambda i: (i, 0)),
            pl.BlockSpec((FO, FO), lambda i: (0, 0)),
            pl.BlockSpec((1, FO), lambda i: (0, 0)),
            pl.BlockSpec((FX, FO), lambda i: (0, 0)),
            pl.BlockSpec((FO, FO), lambda i: (0, 0)),
            pl.BlockSpec((1, FO), lambda i: (0, 0)),
            pl.BlockSpec((FO, FO), lambda i: (0, 0)),
            pl.BlockSpec((1, FO), lambda i: (0, 0)),
            pl.BlockSpec((FO, FO), lambda i: (0, 0)),
            pl.BlockSpec((1, FO), lambda i: (0, 0)),
        ],
        out_specs=pl.BlockSpec((br, FO), lambda i: (i, 0)),
        out_shape=jax.ShapeDtypeStruct((NP, FO), jnp.float32),
    )(s_part, c3, x_pad, wc, bc.reshape(1, FO), wax, wag, ba.reshape(1, FO),
      wb, bb2.reshape(1, FO), wc2, bc2.reshape(1, FO))


def _fold(g, bb_, rm, rv, w, lb):
    s = g * lax.rsqrt(rv + EPS)
    t = bb_ - rm * s
    return w * s[:, None], t @ w + lb


def kernel(x, edge_index, edge_attr, u, batch, g1a, bb1a, rm1a, rv1a, w1a,
           lb1a, g1b, bb1b, rm1b, rv1b, w1b, lb1b, g1c, bb1c, rm1c, rv1c,
           w1c, lb1c, g2a, bb2a, rm2a, rv2a, w2a, lb2a, g2b, bb2b, rm2b,
           rv2b, w2b, lb2b, g2c, bb2c, rm2c, rv2c, w2c, lb2c):
    w1a_f, b1a_f = _fold(g1a, bb1a, rm1a, rv1a, w1a, lb1a)
    w1b_f, b1b_f = _fold(g1b, bb1b, rm1b, rv1b, w1b, lb1b)
    w1c_f, b1c_f = _fold(g1c, bb1c, rm1c, rv1c, w1c, lb1c)
    w2a_f, b2a_f = _fold(g2a, bb2a, rm2a, rv2a, w2a, lb2a)
    w2b_f, b2b_f = _fold(g2b, bb2b, rm2b, rv2b, w2b, lb2b)
    w2c_f, b2c_f = _fold(g2c, bb2c, rm2c, rv2c, w2c, lb2c)
    w1ax, w1ae = w1a_f[:FX], w1a_f[FX:]
    w2ax, w2ag = w2a_f[:FX], w2a_f[FX:]

    row = edge_index[0].astype(jnp.int32)
    col = edge_index[1].astype(jnp.int32)
    x_pad = jnp.pad(x, ((0, NP - NN), (0, 0)))

    p = _tc_pre(x_pad, w1ax, b1a_f)
    gathered = _sc_gather(p, row)
    z2 = _tc_edge(gathered, edge_attr, w1ae, w1b_f, b1b_f)
    s_part, c_part = _sc_scatter(z2, col)
    out = _tc_node(s_part, c_part, x_pad, w1c_f, b1c_f, w2ax, w2ag, b2a_f,
                   w2b_f, b2b_f, w2c_f, b2c_f)
    return out[:NN]
